# Initial kernel scaffold; baseline (speedup 1.0000x reference)
import jax
import jax.numpy as jnp
from jax.experimental import pallas as pl


def kernel(hidden_states, mask, Wq, Wk, Wv, Wo):
    raise NotImplementedError("write your pallas kernel here")


# speedup vs baseline: 1.3887x; 1.3887x over previous
import jax
import jax.numpy as jnp
import numpy as np
from jax.experimental import pallas as pl

_T, _B, _D = 2048, 32, 768
_H, _DK = 12, 64
_TILE_T = 8
_R = _TILE_T * _B


def _mha_body(x_ref, wq_ref, wk_ref, wv_ref, wo_ref, bias_ref, out_ref):
    x = x_ref[...]
    q = jnp.dot(x, wq_ref[...], preferred_element_type=jnp.float32)
    k = jnp.dot(x, wk_ref[...], preferred_element_type=jnp.float32)
    v = jnp.dot(x, wv_ref[...], preferred_element_type=jnp.float32)
    bias = bias_ref[...]
    scale = np.float32(1.0 / np.sqrt(_DK))
    outs = []
    for h in range(_H):
        sl = slice(h * _DK, (h + 1) * _DK)
        qh = q[:, sl] * scale
        kh = k[:, sl]
        vh = v[:, sl]
        s = jax.lax.dot_general(
            qh, kh, (((1,), (1,)), ((), ())),
            preferred_element_type=jnp.float32)
        s = s + bias
        m = jnp.max(s, axis=-1, keepdims=True)
        e = jnp.exp(s - m)
        p = e / jnp.sum(e, axis=-1, keepdims=True)
        outs.append(jnp.dot(p, vh, preferred_element_type=jnp.float32))
    o = jnp.concatenate(outs, axis=-1)
    out_ref[...] = jnp.dot(o, wo_ref[...], preferred_element_type=jnp.float32)


def _group_bias(mask):
    conn = (mask[:, 0] == 0.0).any(axis=0)
    conn = conn | jnp.eye(_B, dtype=bool)
    n_iter = max(1, int(np.ceil(np.log2(max(_B, 2)))))
    for _ in range(n_iter):
        conn = conn | jnp.any(conn[:, :, None] & conn[None, :, :], axis=1)
    gids = jnp.min(jnp.where(conn, jnp.arange(_B), _B), axis=1)
    allow0 = (gids[:, None] == gids[None, :]) | jnp.eye(_B, dtype=bool)
    rb = jnp.arange(_R) % _B
    rt = jnp.arange(_R) // _B
    allow_big = allow0[rb[:, None], rb[None, :]] & (rt[:, None] == rt[None, :])
    return jnp.where(allow_big, 0.0, -1e30).astype(jnp.float32)


def kernel(hidden_states, mask, Wq, Wk, Wv, Wo):
    bias = _group_bias(mask)
    x2 = hidden_states.reshape(_T * _B, _D)
    wspec = pl.BlockSpec((_D, _D), lambda i: (0, 0))
    out = pl.pallas_call(
        _mha_body,
        grid=(_T // _TILE_T,),
        in_specs=[
            pl.BlockSpec((_R, _D), lambda i: (i, 0)),
            wspec, wspec, wspec, wspec,
            pl.BlockSpec((_R, _R), lambda i: (0, 0)),
        ],
        out_specs=pl.BlockSpec((_R, _D), lambda i: (i, 0)),
        out_shape=jax.ShapeDtypeStruct((_T * _B, _D), jnp.float32),
    )(x2, Wq.T, Wk.T, Wv.T, Wo.T, bias)
    return out.reshape(_T, _B, _D)
